# baseline (speedup 1.0000x reference)
import functools

import jax
import jax.numpy as jnp
from jax import lax
from jax.experimental import pallas as pl
from jax.experimental.pallas import tpu as pltpu
from jax.experimental.pallas import tpu_sc as plsc

N_FEATURES = 26
INPUT_DIM = 101
OUT_DIM = 3
BATCH = 16384
ROW = N_FEATURES * OUT_DIM
NUM_WORKERS = 32
COLS_PER_W = BATCH // NUM_WORKERS
NVEC = COLS_PER_W // 16


def _sc_embed(idx_t, tab_t):
    mesh = plsc.VectorSubcoreMesh(core_axis_name="c", subcore_axis_name="s")

    @functools.partial(
        pl.kernel,
        mesh=mesh,
        out_type=jax.ShapeDtypeStruct((ROW, BATCH), jnp.float32),
        compiler_params=pltpu.CompilerParams(
            needs_layout_passes=False,
            disable_bounds_checks=True,
        ),
        scratch_types=[
            pltpu.VMEM((N_FEATURES, COLS_PER_W), jnp.int32),
            pltpu.VMEM((OUT_DIM, N_FEATURES, INPUT_DIM), jnp.float32),
            pltpu.VMEM((ROW, COLS_PER_W), jnp.float32),
            pltpu.SemaphoreType.DMA,
            pltpu.SemaphoreType.DMA,
            pltpu.SemaphoreType.DMA,
            pltpu.SemaphoreType.DMA,
        ],
    )
    def k(idx_hbm, tab_hbm, out_hbm, idx_v, tab_v, out_v,
          s_idx1, s_idx2, s_tab, s_out):
        wid = lax.axis_index("s") * 2 + lax.axis_index("c")
        base = wid * COLS_PER_W
        half = COLS_PER_W // 2

        h_idx1 = pltpu.async_copy(
            idx_hbm.at[:, pl.ds(base, half)], idx_v.at[:, pl.ds(0, half)], s_idx1)
        h_tab = pltpu.async_copy(tab_hbm, tab_v, s_tab)
        h_idx2 = pltpu.async_copy(
            idx_hbm.at[:, pl.ds(base + half, half)],
            idx_v.at[:, pl.ds(half, half)], s_idx2)

        def compute(lo, hi):
            @plsc.parallel_loop(lo, hi, unroll=1)
            def vec(v):
                c0 = v * 16
                for f in range(N_FEATURES):
                    iv = idx_v[f, pl.ds(c0, 16)]
                    for d in range(OUT_DIM):
                        e = plsc.load_gather(tab_v.at[d, f], [iv])
                        out_v[OUT_DIM * f + d, pl.ds(c0, 16)] = e

        h_idx1.wait()
        h_tab.wait()
        with jax.named_scope("compute1"):
            compute(0, NVEC // 2)
        h_out1 = pltpu.async_copy(
            out_v.at[:, pl.ds(0, half)], out_hbm.at[:, pl.ds(base, half)], s_out)
        h_idx2.wait()
        with jax.named_scope("compute2"):
            compute(NVEC // 2, NVEC)
        with jax.named_scope("out_dma"):
            pltpu.sync_copy(out_v.at[:, pl.ds(half, half)],
                            out_hbm.at[:, pl.ds(base + half, half)])
        h_out1.wait()

    return k(idx_t, tab_t)


def kernel(inputs, tables):
    out_t = _sc_embed(inputs.T, tables.transpose(2, 0, 1))
    return out_t.T


# speedup vs baseline: 1.0802x; 1.0802x over previous
import functools

import jax
import jax.numpy as jnp
from jax import lax
from jax.experimental import pallas as pl
from jax.experimental.pallas import tpu as pltpu
from jax.experimental.pallas import tpu_sc as plsc

N_FEATURES = 26
INPUT_DIM = 101
OUT_DIM = 3
BATCH = 16384
ROW = N_FEATURES * OUT_DIM
NUM_WORKERS = 32
COLS_PER_W = BATCH // NUM_WORKERS
NVEC = COLS_PER_W // 16


def _sc_embed(idx_t, tab_t):
    mesh = plsc.VectorSubcoreMesh(core_axis_name="c", subcore_axis_name="s")

    @functools.partial(
        pl.kernel,
        mesh=mesh,
        out_type=jax.ShapeDtypeStruct((ROW, BATCH), jnp.float32),
        compiler_params=pltpu.CompilerParams(
            needs_layout_passes=False,
            disable_bounds_checks=True,
        ),
        scratch_types=[
            pltpu.VMEM((N_FEATURES, COLS_PER_W), jnp.int32),
            pltpu.VMEM((OUT_DIM, N_FEATURES, INPUT_DIM), jnp.float32),
            pltpu.VMEM((ROW, COLS_PER_W), jnp.float32),
            pltpu.SemaphoreType.DMA,
            pltpu.SemaphoreType.DMA,
        ],
    )
    def k(idx_hbm, tab_hbm, out_hbm, idx_v, tab_v, out_v, s_idx, s_tab):
        wid = lax.axis_index("s") * 2 + lax.axis_index("c")
        base = wid * COLS_PER_W

        h_idx = pltpu.async_copy(
            idx_hbm.at[:, pl.ds(base, COLS_PER_W)], idx_v, s_idx)
        h_tab = pltpu.async_copy(tab_hbm, tab_v, s_tab)
        h_idx.wait()
        h_tab.wait()

        with jax.named_scope("compute"):
            @plsc.parallel_loop(0, NVEC, unroll=1)
            def vec(v):
                c0 = v * 16
                for f in range(N_FEATURES):
                    iv = idx_v[f, pl.ds(c0, 16)]
                    for d in range(OUT_DIM):
                        e = plsc.load_gather(tab_v.at[d, f], [iv])
                        out_v[OUT_DIM * f + d, pl.ds(c0, 16)] = e

        with jax.named_scope("out_dma"):
            pltpu.sync_copy(out_v, out_hbm.at[:, pl.ds(base, COLS_PER_W)])

    return k(idx_t, tab_t)


def kernel(inputs, tables):
    out_t = _sc_embed(inputs.T, tables.transpose(2, 0, 1))
    return out_t.T
